# baseline (speedup 1.0000x reference)
import functools

import jax
import jax.numpy as jnp
from jax import lax
from jax.experimental import pallas as pl
from jax.experimental.pallas import tpu as pltpu
from jax.experimental.pallas import tpu_sc as plsc

NUM_EMB = 1000001
D = 64
B = 4096
L = 200
HALF = 100
NW = 32
BAGS_PER_W = B // NW


def _bow_sc(table, ids2):
    mesh = plsc.VectorSubcoreMesh(core_axis_name="c", subcore_axis_name="s")

    @functools.partial(
        pl.kernel,
        mesh=mesh,
        out_type=jax.ShapeDtypeStruct((B, D), jnp.float32),
        compiler_params=pltpu.CompilerParams(use_tc_tiling_on_sc=False),
        scratch_types=[
            pltpu.VMEM((2 * BAGS_PER_W, HALF), jnp.int32),
            pltpu.VMEM((L, D), jnp.float32),
            pltpu.VMEM((BAGS_PER_W, D), jnp.float32),
            pltpu.SemaphoreType.DMA,
        ],
    )
    def k(table_hbm, ids_hbm, out_hbm, idx_v, rows_v, out_v, sem):
        wid = lax.axis_index("s") * 2 + lax.axis_index("c")
        base = wid * BAGS_PER_W

        pltpu.sync_copy(ids_hbm.at[pl.ds(2 * base, 2 * BAGS_PER_W)], idx_v)

        def bag_body(b, carry):
            cp0 = pltpu.async_copy(
                table_hbm.at[idx_v.at[2 * b]], rows_v.at[pl.ds(0, HALF)], sem)
            cp1 = pltpu.async_copy(
                table_hbm.at[idx_v.at[2 * b + 1]], rows_v.at[pl.ds(HALF, HALF)],
                sem)
            cp0.wait()
            cp1.wait()

            def racc(r, acc):
                out = []
                for j in range(4):
                    a = acc[j]
                    for u in range(4):
                        a = a + rows_v[4 * r + u, pl.ds(16 * j, 16)]
                    out.append(a)
                return tuple(out)

            acc = lax.fori_loop(
                0, L // 4, racc,
                tuple(jnp.zeros((16,), jnp.float32) for _ in range(4)))
            for j in range(4):
                out_v[b, pl.ds(16 * j, 16)] = acc[j]
            return carry

        lax.fori_loop(0, BAGS_PER_W, bag_body, 0)
        pltpu.sync_copy(out_v, out_hbm.at[pl.ds(base, BAGS_PER_W)])

    return k(table, ids2)


def kernel(input_ids, table):
    ids2 = input_ids.astype(jnp.int32).reshape(2 * B, HALF)
    return _bow_sc(table, ids2)


# speedup vs baseline: 1.1361x; 1.1361x over previous
import functools

import jax
import jax.numpy as jnp
from jax import lax
from jax.experimental import pallas as pl
from jax.experimental.pallas import tpu as pltpu
from jax.experimental.pallas import tpu_sc as plsc

NUM_EMB = 1000001
D = 64
B = 4096
L = 200
HALF = 100
NW = 32
BAGS_PER_W = B // NW


def _bow_sc(table, ids2):
    mesh = plsc.VectorSubcoreMesh(core_axis_name="c", subcore_axis_name="s")

    @functools.partial(
        pl.kernel,
        mesh=mesh,
        out_type=jax.ShapeDtypeStruct((B, D), jnp.float32),
        compiler_params=pltpu.CompilerParams(use_tc_tiling_on_sc=False),
        scratch_types=[
            pltpu.VMEM((2 * BAGS_PER_W, HALF), jnp.int32),
            pltpu.VMEM((L, D), jnp.float32),
            pltpu.VMEM((L, D), jnp.float32),
            pltpu.VMEM((BAGS_PER_W, D), jnp.float32),
            pltpu.SemaphoreType.DMA,
            pltpu.SemaphoreType.DMA,
        ],
    )
    def k(table_hbm, ids_hbm, out_hbm, idx_v, rows0, rows1, out_v, sem0, sem1):
        wid = lax.axis_index("s") * 2 + lax.axis_index("c")
        base = wid * BAGS_PER_W

        pltpu.sync_copy(ids_hbm.at[pl.ds(2 * base, 2 * BAGS_PER_W)], idx_v)

        def start_gather(b, rows_v, sem):
            pltpu.async_copy(
                table_hbm.at[idx_v.at[2 * b]], rows_v.at[pl.ds(0, HALF)], sem)
            pltpu.async_copy(
                table_hbm.at[idx_v.at[2 * b + 1]],
                rows_v.at[pl.ds(HALF, HALF)], sem)

        def wait_gather(b, rows_v, sem):
            pltpu.make_async_copy(
                table_hbm.at[idx_v.at[2 * b]], rows_v.at[pl.ds(0, HALF)],
                sem).wait()
            pltpu.make_async_copy(
                table_hbm.at[idx_v.at[2 * b + 1]],
                rows_v.at[pl.ds(HALF, HALF)], sem).wait()

        def accumulate(b, rows_v):
            def racc(r, acc):
                out = []
                for j in range(4):
                    a = acc[j]
                    for u in range(4):
                        a = a + rows_v[4 * r + u, pl.ds(16 * j, 16)]
                    out.append(a)
                return tuple(out)

            acc = lax.fori_loop(
                0, L // 4, racc,
                tuple(jnp.zeros((16,), jnp.float32) for _ in range(4)))
            for j in range(4):
                out_v[b, pl.ds(16 * j, 16)] = acc[j]

        start_gather(0, rows0, sem0)

        def pair_body(p, carry):
            b0 = 2 * p
            start_gather(b0 + 1, rows1, sem1)
            wait_gather(b0, rows0, sem0)
            accumulate(b0, rows0)

            @pl.when(p < BAGS_PER_W // 2 - 1)
            def _():
                start_gather(b0 + 2, rows0, sem0)

            wait_gather(b0 + 1, rows1, sem1)
            accumulate(b0 + 1, rows1)
            return carry

        lax.fori_loop(0, BAGS_PER_W // 2, pair_body, 0)
        pltpu.sync_copy(out_v, out_hbm.at[pl.ds(base, BAGS_PER_W)])

    return k(table, ids2)


def kernel(input_ids, table):
    ids2 = input_ids.astype(jnp.int32).reshape(2 * B, HALF)
    return _bow_sc(table, ids2)


# speedup vs baseline: 1.3932x; 1.2263x over previous
import functools

import jax
import jax.numpy as jnp
from jax import lax
from jax.experimental import pallas as pl
from jax.experimental.pallas import tpu as pltpu
from jax.experimental.pallas import tpu_sc as plsc

NUM_EMB = 1000001
D = 64
B = 4096
L = 200
NW = 32
BAGS_PER_W = B // NW
NG = L // 16
TAIL = L - 16 * NG


def _bow_sc(table, ids):
    mesh = plsc.VectorSubcoreMesh(core_axis_name="c", subcore_axis_name="s")

    @functools.partial(
        pl.kernel,
        mesh=mesh,
        out_type=jax.ShapeDtypeStruct((B, D), jnp.float32),
        scratch_types=[
            pltpu.VMEM((BAGS_PER_W, L), jnp.int32),
            pltpu.VMEM((L, D), jnp.float32),
            pltpu.VMEM((L, D), jnp.float32),
            pltpu.VMEM((BAGS_PER_W, D), jnp.float32),
            pltpu.SemaphoreType.DMA,
            pltpu.SemaphoreType.DMA,
        ],
    )
    def k(table_hbm, ids_hbm, out_hbm, idx_v, rows0, rows1, out_v, sem0, sem1):
        wid = lax.axis_index("s") * 2 + lax.axis_index("c")
        base = wid * BAGS_PER_W

        pltpu.sync_copy(ids_hbm.at[pl.ds(base, BAGS_PER_W)], idx_v)

        def enq_chunk(b, r0, lanes, rows_v, sem):
            iv = idx_v[b, pl.ds(r0, 16)]
            for u in range(16 - lanes, 16):
                i = iv[u]
                pltpu.async_copy(
                    table_hbm.at[pl.ds(i, 1)],
                    rows_v.at[pl.ds(r0 + u, 1)], sem)

        def enqueue_bag(b, rows_v, sem):
            def enq(g, carry):
                enq_chunk(b, 16 * g, 16, rows_v, sem)
                return carry

            lax.fori_loop(0, NG, enq, 0)
            enq_chunk(b, L - 16, TAIL, rows_v, sem)

        def wait_bag(rows_v, sem):
            pltpu.make_async_copy(
                table_hbm.at[pl.ds(0, L)], rows_v, sem).wait()

        def acc_store(b, acc):
            for j in range(4):
                out_v[b, pl.ds(16 * j, 16)] = acc[j]

        def zeros4():
            return tuple(jnp.zeros((16,), jnp.float32) for _ in range(4))

        def acc_rows(acc, rows_v, r0, n):
            out = list(acc)
            for u in range(n):
                for j in range(4):
                    out[j] = out[j] + rows_v[r0 + u, pl.ds(16 * j, 16)]
            return tuple(out)

        def accumulate(b, rows_v):
            def racc(g, acc):
                return acc_rows(acc, rows_v, 16 * g, 16)

            acc = lax.fori_loop(0, NG, racc, zeros4())
            acc = acc_rows(acc, rows_v, 16 * NG, TAIL)
            acc_store(b, acc)

        def acc_plus_enqueue(b_acc, rows_acc, b_enq, rows_enq, sem_enq):
            def body(g, acc):
                enq_chunk(b_enq, 16 * g, 16, rows_enq, sem_enq)
                return acc_rows(acc, rows_acc, 16 * g, 16)

            acc = lax.fori_loop(0, NG, body, zeros4())
            enq_chunk(b_enq, L - 16, TAIL, rows_enq, sem_enq)
            acc = acc_rows(acc, rows_acc, 16 * NG, TAIL)
            acc_store(b_acc, acc)

        enqueue_bag(0, rows0, sem0)

        def pair_body(p, carry):
            b0 = 2 * p
            wait_bag(rows0, sem0)
            acc_plus_enqueue(b0, rows0, b0 + 1, rows1, sem1)
            wait_bag(rows1, sem1)
            acc_plus_enqueue(b0 + 1, rows1, b0 + 2, rows0, sem0)
            return carry

        lax.fori_loop(0, BAGS_PER_W // 2 - 1, pair_body, 0)

        wait_bag(rows0, sem0)
        acc_plus_enqueue(BAGS_PER_W - 2, rows0, BAGS_PER_W - 1, rows1, sem1)
        wait_bag(rows1, sem1)
        accumulate(BAGS_PER_W - 1, rows1)

        pltpu.sync_copy(out_v, out_hbm.at[pl.ds(base, BAGS_PER_W)])

    return k(table, ids)


def kernel(input_ids, table):
    return _bow_sc(table, input_ids.astype(jnp.int32))


# speedup vs baseline: 1.3951x; 1.0014x over previous
import functools

import jax
import jax.numpy as jnp
from jax import lax
from jax.experimental import pallas as pl
from jax.experimental.pallas import tpu as pltpu
from jax.experimental.pallas import tpu_sc as plsc

NUM_EMB = 1000001
D = 64
B = 4096
L = 200
HALF = 100
NW = 32
BAGS_PER_W = B // NW

CHUNK = 2048
NBLK = 489
VPAD = NBLK * CHUNK
PACK_ROWS = VPAD // 2


def _repack_tc(table_t):
    def body(x_ref, o_ref):
        t = jnp.transpose(x_ref[...], (1, 0))
        o_ref[...] = jnp.concatenate(
            [t[: CHUNK // 2], t[CHUNK // 2:]], axis=1)

    return pl.pallas_call(
        body,
        grid=(NBLK,),
        in_specs=[pl.BlockSpec((D, CHUNK), lambda g: (0, g))],
        out_specs=pl.BlockSpec((CHUNK // 2, 2 * D), lambda g: (g, 0)),
        out_shape=jax.ShapeDtypeStruct((PACK_ROWS, 2 * D), jnp.float32),
    )(table_t)


def _bow_sc(table, ids2):
    mesh = plsc.VectorSubcoreMesh(core_axis_name="c", subcore_axis_name="s")

    @functools.partial(
        pl.kernel,
        mesh=mesh,
        out_type=jax.ShapeDtypeStruct((B, D), jnp.float32),
        compiler_params=pltpu.CompilerParams(use_tc_tiling_on_sc=False),
        scratch_types=[
            pltpu.VMEM((2 * BAGS_PER_W, HALF), jnp.int32),
            pltpu.VMEM((L, D), jnp.float32),
            pltpu.VMEM((L, D), jnp.float32),
            pltpu.VMEM((BAGS_PER_W, D), jnp.float32),
            pltpu.SemaphoreType.DMA,
            pltpu.SemaphoreType.DMA,
        ],
    )
    def k(table_hbm, ids_hbm, out_hbm, idx_v, rows0, rows1, out_v, sem0, sem1):
        wid = lax.axis_index("s") * 2 + lax.axis_index("c")
        base = wid * BAGS_PER_W

        pltpu.sync_copy(ids_hbm.at[pl.ds(2 * base, 2 * BAGS_PER_W)], idx_v)

        def start_gather(b, rows_v, sem):
            pltpu.async_copy(
                table_hbm.at[idx_v.at[2 * b]], rows_v.at[pl.ds(0, HALF)], sem)
            pltpu.async_copy(
                table_hbm.at[idx_v.at[2 * b + 1]],
                rows_v.at[pl.ds(HALF, HALF)], sem)

        def wait_gather(b, rows_v, sem):
            pltpu.make_async_copy(
                table_hbm.at[idx_v.at[2 * b]], rows_v.at[pl.ds(0, HALF)],
                sem).wait()
            pltpu.make_async_copy(
                table_hbm.at[idx_v.at[2 * b + 1]],
                rows_v.at[pl.ds(HALF, HALF)], sem).wait()

        def accumulate(b, rows_v):
            def racc(r, acc):
                out = []
                for j in range(4):
                    a = acc[j]
                    for u in range(4):
                        a = a + rows_v[4 * r + u, pl.ds(16 * j, 16)]
                    out.append(a)
                return tuple(out)

            acc = lax.fori_loop(
                0, L // 4, racc,
                tuple(jnp.zeros((16,), jnp.float32) for _ in range(4)))
            for j in range(4):
                out_v[b, pl.ds(16 * j, 16)] = acc[j]

        start_gather(0, rows0, sem0)

        def pair_body(p, carry):
            b0 = 2 * p
            start_gather(b0 + 1, rows1, sem1)
            wait_gather(b0, rows0, sem0)
            accumulate(b0, rows0)

            @pl.when(p < BAGS_PER_W // 2 - 1)
            def _():
                start_gather(b0 + 2, rows0, sem0)

            wait_gather(b0 + 1, rows1, sem1)
            accumulate(b0 + 1, rows1)
            return carry

        lax.fori_loop(0, BAGS_PER_W // 2, pair_body, 0)
        pltpu.sync_copy(out_v, out_hbm.at[pl.ds(base, BAGS_PER_W)])

    return k(table, ids2)


def kernel(input_ids, table):
    packed = _repack_tc(table.T)
    table_lin = packed.reshape(VPAD, D)
    i = input_ids.astype(jnp.int32)
    c = i & (CHUNK - 1)
    r = (i & ~(CHUNK - 1)) | ((c & (CHUNK // 2 - 1)) << 1) | (c >> 10)
    ids2 = r.reshape(2 * B, HALF)
    return _bow_sc(table_lin, ids2)
